# baseline (speedup 1.0000x reference)
import jax
import jax.numpy as jnp
from jax.experimental import pallas as pl
from jax.experimental.pallas import tpu as pltpu

_VOCAB = 1_000_000
_EMB = 64
_CHUNK = 55_552
_NCH = 18
_TAIL_BASE = _NCH * _CHUNK
_TAIL = _VOCAB - _TAIL_BASE
_K = 4
_L = 2


def _tc_body(in_hbm, out_hbm, buf, tail_buf, in_sems, out_sems, tail_sems):
    def in_copy(j):
        s = j % _K
        return pltpu.make_async_copy(
            in_hbm.at[:, pl.ds(j * _CHUNK, _CHUNK)], buf.at[s], in_sems.at[s]
        )

    def out_copy(j):
        s = j % _K
        return pltpu.make_async_copy(
            buf.at[s], out_hbm.at[:, pl.ds(j * _CHUNK, _CHUNK)], out_sems.at[s]
        )

    t_in = pltpu.make_async_copy(
        in_hbm.at[:, pl.ds(_TAIL_BASE, _TAIL)], tail_buf, tail_sems.at[0]
    )
    t_out = pltpu.make_async_copy(
        tail_buf, out_hbm.at[:, pl.ds(_TAIL_BASE, _TAIL)], tail_sems.at[1]
    )
    t_in.start()

    waited = set()
    for j in range(_L):
        in_copy(j).start()
    t_in.wait()
    t_out.start()
    for j in range(_NCH):
        in_copy(j).wait()
        out_copy(j).start()
        nxt = j + _L
        if nxt < _NCH:
            prev = nxt - _K
            if prev >= 0:
                out_copy(prev).wait()
                waited.add(prev)
            in_copy(nxt).start()
    for j in range(_NCH):
        if j not in waited:
            out_copy(j).wait()
    t_out.wait()


def kernel(lang, W_emb):
    del lang
    W_t = W_emb.T
    out = pl.pallas_call(
        _tc_body,
        in_specs=[pl.BlockSpec(memory_space=pltpu.MemorySpace.HBM)],
        out_specs=pl.BlockSpec(memory_space=pltpu.MemorySpace.HBM),
        out_shape=jax.ShapeDtypeStruct((_EMB, _VOCAB), jnp.float32),
        scratch_shapes=[
            pltpu.VMEM((_K, _EMB, _CHUNK), jnp.float32),
            pltpu.VMEM((_EMB, _TAIL), jnp.float32),
            pltpu.SemaphoreType.DMA((_K,)),
            pltpu.SemaphoreType.DMA((_K,)),
            pltpu.SemaphoreType.DMA((2,)),
        ],
    )(W_t)
    return out.T


# speedup vs baseline: 1.0017x; 1.0017x over previous
import jax
import jax.numpy as jnp
from jax.experimental import pallas as pl
from jax.experimental.pallas import tpu as pltpu

_VOCAB = 1_000_000
_EMB = 64
_CHUNK = 47_616
_NCH = 21
_TAIL_BASE = _NCH * _CHUNK
_TAIL = _VOCAB - _TAIL_BASE
_K = 4
_L = 2


def _tc_body(in_hbm, out_hbm, buf, tail_buf, in_sems, out_sems, tail_sems):
    def in_copy(j):
        s = j % _K
        return pltpu.make_async_copy(
            in_hbm.at[:, pl.ds(j * _CHUNK, _CHUNK)], buf.at[s], in_sems.at[s]
        )

    def out_copy(j):
        s = j % _K
        return pltpu.make_async_copy(
            buf.at[s], out_hbm.at[:, pl.ds(j * _CHUNK, _CHUNK)], out_sems.at[s]
        )

    t_in = pltpu.make_async_copy(
        in_hbm.at[:, pl.ds(_TAIL_BASE, _TAIL)], tail_buf, tail_sems.at[0]
    )
    t_out = pltpu.make_async_copy(
        tail_buf, out_hbm.at[:, pl.ds(_TAIL_BASE, _TAIL)], tail_sems.at[1]
    )
    t_in.start()

    waited = set()
    for j in range(_L):
        in_copy(j).start()
    t_in.wait()
    t_out.start()
    for j in range(_NCH):
        in_copy(j).wait()
        out_copy(j).start()
        nxt = j + _L
        if nxt < _NCH:
            prev = nxt - _K
            if prev >= 0:
                out_copy(prev).wait()
                waited.add(prev)
            in_copy(nxt).start()
    for j in range(_NCH):
        if j not in waited:
            out_copy(j).wait()
    t_out.wait()


def kernel(lang, W_emb):
    del lang
    W_t = W_emb.T
    out = pl.pallas_call(
        _tc_body,
        in_specs=[pl.BlockSpec(memory_space=pltpu.MemorySpace.HBM)],
        out_specs=pl.BlockSpec(memory_space=pltpu.MemorySpace.HBM),
        out_shape=jax.ShapeDtypeStruct((_EMB, _VOCAB), jnp.float32),
        scratch_shapes=[
            pltpu.VMEM((_K, _EMB, _CHUNK), jnp.float32),
            pltpu.VMEM((_EMB, _TAIL), jnp.float32),
            pltpu.SemaphoreType.DMA((_K,)),
            pltpu.SemaphoreType.DMA((_K,)),
            pltpu.SemaphoreType.DMA((2,)),
        ],
    )(W_t)
    return out.T
